# baseline (speedup 1.0000x reference)
import functools

import jax
import jax.numpy as jnp
from jax import lax
from jax.experimental import pallas as pl
from jax.experimental.pallas import tpu as pltpu

_EPS = 1e-5
_VMEM_LIMIT = 64 * 1024 * 1024
_GRAM_DN = (((0,), (0,)), ((), ()))


def _cparams():
    return pltpu.CompilerParams(
        dimension_semantics=("parallel",),
        vmem_limit_bytes=_VMEM_LIMIT)


def _rup16(v):
    return ((v + 15) // 16) * 16


def _ik_gram_affine(gsum, usum, wmat, gamma, beta, count):
    mean = jnp.dot(usum, wmat, preferred_element_type=jnp.float32) / count
    gw = jnp.dot(gsum, wmat, preferred_element_type=jnp.float32)
    q = jnp.sum(wmat * gw, axis=0, keepdims=True) / count
    var = jnp.maximum(q - mean * mean, 0.0)
    scale = gamma * lax.rsqrt(var + _EPS)
    shift = beta - mean * scale
    return scale, shift


def _ik_direct_affine(ssum, qsum, gamma, beta, count):
    mean = ssum / count
    var = jnp.maximum(qsum / count - mean * mean, 0.0)
    scale = gamma * lax.rsqrt(var + _EPS)
    shift = beta - mean * scale
    return scale, shift


def _pass_a_kernel(x_ref, x16_ref, sa_ref):
    xb = x_ref[...]
    x16 = xb.astype(jnp.bfloat16)
    x16_ref[...] = x16
    g = lax.dot_general(x16, x16, _GRAM_DN,
                        preferred_element_type=jnp.float32)
    u = jnp.sum(xb, axis=0, keepdims=True)
    sa_ref[...] = jnp.concatenate([g, u], axis=0)[None]


def _pass_a(x2d, tm):
    m, c0 = x2d.shape
    nt = m // tm
    return pl.pallas_call(
        _pass_a_kernel,
        grid=(nt,),
        in_specs=[pl.BlockSpec((tm, c0), lambda i: (i, 0))],
        out_specs=(pl.BlockSpec((tm, c0), lambda i: (i, 0)),
                   pl.BlockSpec((1, c0 + 1, c0), lambda i: (i, 0, 0))),
        out_shape=(jax.ShapeDtypeStruct((m, c0), jnp.bfloat16),
                   jax.ShapeDtypeStruct((nt, c0 + 1, c0), jnp.float32)),
        compiler_params=_cparams(),
        cost_estimate=pl.CostEstimate(
            flops=2 * m * c0 * c0, transcendentals=0,
            bytes_accessed=4 * m * c0 + 2 * m * c0),
    )(x2d)


def _pass_b_kernel(x_ref, w1_ref, w2_ref, sa_ref, gb_ref,
                   y_ref, sb_ref, p_ref, q1_ref, q7_ref, col_ref, *, count):
    nb, h, w, c0 = x_ref.shape
    c = w1_ref.shape[1]
    wp = _rup16(w + 2)
    base = wp + 8
    hw = h * wp
    psz = p_ref.shape[0] // nb
    qsz = q1_ref.shape[0] // nb
    cout = w2_ref.shape[1]

    ta = jnp.sum(sa_ref[...], axis=0)
    sc1, sh1 = _ik_gram_affine(ta[:c0], ta[c0:c0 + 1], w1_ref[...],
                               gb_ref[0:1, :c], gb_ref[1:2, :c], count)

    h1 = jnp.dot(x_ref[...].reshape(nb * h * w, c0),
                 w1_ref[...].astype(jnp.bfloat16),
                 preferred_element_type=jnp.float32)
    a3 = jnp.maximum(h1 * sc1 + sh1, 0.0).reshape(nb * h, w, c)

    for j in range(nb):
        pb = j * psz
        p_ref[pb:pb + base, :] = jnp.zeros((base, c), jnp.float32)
        tail = base + (h - 1) * wp + w
        p_ref[pb + tail:pb + psz, :] = jnp.zeros((psz - tail, c),
                                                 jnp.float32)
        zgap = jnp.zeros((wp - w, c), jnp.float32)
        for hh in range(h - 1):
            p_ref[pb + base + hh * wp + w:pb + base + (hh + 1) * wp, :] = \
                zgap
        for hh in range(h):
            p_ref[pb + base + hh * wp:pb + base + hh * wp + w, :] = \
                a3[j * h + hh]

        q1_ref[j * qsz:(j + 1) * qsz, :] = p_ref[pb + 1:pb + qsz + 1, :]
        q7_ref[j * qsz:(j + 1) * qsz, :] = p_ref[pb + 7:pb + qsz + 7, :]

        for kw in range(3):
            src = (q7_ref, p_ref, q1_ref)[kw]
            sb = (j * qsz, pb, j * qsz)[kw]
            start = sb + base - wp + (kw - 1) - (7, 0, 1)[kw]
            v16 = src[start:start + hw + 2 * wp, :].astype(jnp.bfloat16)
            for kh in range(3):
                t = kh * 3 + kw
                col_ref[j * hw:(j + 1) * hw, t * c:(t + 1) * c] = \
                    v16[kh * wp:kh * wp + hw]

    y = jnp.dot(col_ref[...], w2_ref[...].astype(jnp.bfloat16),
                preferred_element_type=jnp.float32)
    y4 = y.reshape(nb, h, wp, cout)[:, :, :w, :]
    y_ref[...] = y4.astype(jnp.bfloat16)
    sq = []
    for j in range(nb):
        yf = y4[j].reshape(h * w, cout)
        s = jnp.sum(yf, axis=0, keepdims=True)
        q = jnp.sum(yf * yf, axis=0, keepdims=True)
        sq.append(jnp.concatenate([s, q], axis=0))
    sb_ref[...] = jnp.stack(sq, axis=0)


def _pass_b(x4d, w1, w2f, st_a, gb, count):
    n, h, w, c0 = x4d.shape
    c = w1.shape[1]
    cout = w2f.shape[1]
    nta, c0p1, c0a = st_a.shape
    nb = 2 if n % 2 == 0 else 1
    wp = _rup16(w + 2)
    base = wp + 8
    hw = h * wp
    qsz = base + wp + hw
    psz = qsz + 8
    return pl.pallas_call(
        functools.partial(_pass_b_kernel, count=count),
        grid=(n // nb,),
        in_specs=[pl.BlockSpec((nb, h, w, c0), lambda i: (i, 0, 0, 0)),
                  pl.BlockSpec((c0, c), lambda i: (0, 0)),
                  pl.BlockSpec((9 * c, cout), lambda i: (0, 0)),
                  pl.BlockSpec((nta, c0p1, c0a), lambda i: (0, 0, 0)),
                  pl.BlockSpec(gb.shape, lambda i: (0, 0))],
        out_specs=(pl.BlockSpec((nb, h, w, cout), lambda i: (i, 0, 0, 0)),
                   pl.BlockSpec((nb, 2, cout), lambda i: (i, 0, 0))),
        out_shape=(jax.ShapeDtypeStruct((n, h, w, cout), jnp.bfloat16),
                   jax.ShapeDtypeStruct((n, 2, cout), jnp.float32)),
        scratch_shapes=[pltpu.VMEM((nb * psz, c), jnp.float32),
                        pltpu.VMEM((nb * qsz, c), jnp.float32),
                        pltpu.VMEM((nb * qsz, c), jnp.float32),
                        pltpu.VMEM((nb * hw, 9 * c), jnp.bfloat16)],
        compiler_params=_cparams(),
        cost_estimate=pl.CostEstimate(
            flops=2 * n * hw * c * (9 * cout + c0), transcendentals=0,
            bytes_accessed=2 * (n * h * w * c0 + n * h * w * cout)
                           + 4 * 9 * c * cout),
    )(x4d, w1, w2f, st_a, gb)


def _pass_c_kernel(h2_ref, sb_ref, gb_ref, st_ref, *, count):
    c = h2_ref.shape[1]
    tb = jnp.sum(sb_ref[...], axis=0)
    sc2, sh2 = _ik_direct_affine(tb[0:1], tb[1:2],
                                 gb_ref[2:3, :c], gb_ref[3:4, :c], count)
    t = jnp.maximum(h2_ref[...].astype(jnp.float32) * sc2 + sh2, 0.0)
    t16 = t.astype(jnp.bfloat16)
    g = lax.dot_general(t16, t16, _GRAM_DN,
                        preferred_element_type=jnp.float32)
    u = jnp.sum(t, axis=0, keepdims=True)
    st_ref[...] = jnp.concatenate([g, u], axis=0)[None]


def _pass_c(h2d, st_b, gb, tm, count):
    m, cm = h2d.shape
    nt = m // tm
    n2, two, cmb = st_b.shape
    return pl.pallas_call(
        functools.partial(_pass_c_kernel, count=count),
        grid=(nt,),
        in_specs=[pl.BlockSpec((tm, cm), lambda i: (i, 0)),
                  pl.BlockSpec((n2, two, cmb), lambda i: (0, 0, 0)),
                  pl.BlockSpec(gb.shape, lambda i: (0, 0))],
        out_specs=pl.BlockSpec((1, cm + 1, cm), lambda i: (i, 0, 0)),
        out_shape=jax.ShapeDtypeStruct((nt, cm + 1, cm), jnp.float32),
        compiler_params=_cparams(),
        cost_estimate=pl.CostEstimate(
            flops=2 * m * cm * cm, transcendentals=0,
            bytes_accessed=2 * m * cm),
    )(h2d, st_b, gb)


def _pass_d_kernel(h2_ref, x_ref, w3_ref, ws_ref, sa_ref, sb_ref, st_ref,
                   gb_ref, o_ref, *, count):
    cm = w3_ref.shape[0]
    c0 = ws_ref.shape[0]
    tb = jnp.sum(sb_ref[...], axis=0)
    sc2, sh2 = _ik_direct_affine(tb[0:1], tb[1:2],
                                 gb_ref[2:3, :cm], gb_ref[3:4, :cm], count)
    w3f = w3_ref[...]
    wsf = ws_ref[...]
    tc = jnp.sum(st_ref[...], axis=0)
    sc3, sh3 = _ik_gram_affine(tc[:cm], tc[cm:cm + 1], w3f,
                               gb_ref[4:5, :], gb_ref[5:6, :], count)
    ta = jnp.sum(sa_ref[...], axis=0)
    scs, shs = _ik_gram_affine(ta[:c0], ta[c0:c0 + 1], wsf,
                               gb_ref[6:7, :], gb_ref[7:8, :], count)

    t = jnp.maximum(h2_ref[...].astype(jnp.float32) * sc2 + sh2, 0.0)
    z = jnp.dot(t.astype(jnp.bfloat16), w3f.astype(jnp.bfloat16),
                preferred_element_type=jnp.float32)
    r = jnp.dot(x_ref[...], wsf.astype(jnp.bfloat16),
                preferred_element_type=jnp.float32)
    o = (z * sc3 + sh3) + (r * scs + shs)
    o_ref[...] = jnp.maximum(o, 0.0).astype(o_ref.dtype)


def _pass_d(h2d, x16, w3, ws, st_a, st_b, st_c, gb, out_dtype, tm, count):
    m, cm = h2d.shape
    c0 = x16.shape[1]
    ce = w3.shape[1]
    nt = m // tm
    nta, ap, ac = st_a.shape
    nb, two, cmb = st_b.shape
    ntc, cp, cc = st_c.shape
    return pl.pallas_call(
        functools.partial(_pass_d_kernel, count=count),
        grid=(nt,),
        in_specs=[pl.BlockSpec((tm, cm), lambda i: (i, 0)),
                  pl.BlockSpec((tm, c0), lambda i: (i, 0)),
                  pl.BlockSpec((cm, ce), lambda i: (0, 0)),
                  pl.BlockSpec((c0, ce), lambda i: (0, 0)),
                  pl.BlockSpec((nta, ap, ac), lambda i: (0, 0, 0)),
                  pl.BlockSpec((nb, two, cmb), lambda i: (0, 0, 0)),
                  pl.BlockSpec((ntc, cp, cc), lambda i: (0, 0, 0)),
                  pl.BlockSpec(gb.shape, lambda i: (0, 0))],
        out_specs=pl.BlockSpec((tm, ce), lambda i: (i, 0)),
        out_shape=jax.ShapeDtypeStruct((m, ce), out_dtype),
        compiler_params=_cparams(),
        cost_estimate=pl.CostEstimate(
            flops=2 * m * (cm + c0) * ce, transcendentals=0,
            bytes_accessed=2 * m * cm + 2 * m * c0 + 4 * m * ce),
    )(h2d, x16, w3, ws, st_a, st_b, st_c, gb)


def kernel(x, w1, g1, b1, w2, g2, b2, w3, g3, b3, ws, gs, bs):
    n, h, w, c0 = x.shape
    cm = w1.shape[1]
    ce = w3.shape[1]
    m = n * h * w
    fm = float(m)
    tm = 16384 if m % 16384 == 0 else m

    x2d = x.reshape(m, c0)
    w2f = w2.reshape(9 * cm, cm)

    pad = lambda v: jnp.pad(v.reshape(1, -1),
                            ((0, 0), (0, ce - v.shape[-1])))
    gb = jnp.concatenate([pad(g1), pad(b1), pad(g2), pad(b2),
                          g3.reshape(1, -1), b3.reshape(1, -1),
                          gs.reshape(1, -1), bs.reshape(1, -1)], axis=0)

    x16, st_a = _pass_a(x2d, tm)
    h2, st_b = _pass_b(x16.reshape(n, h, w, c0), w1, w2f, st_a, gb, fm)
    h2d = h2.reshape(m, cm)
    st_c = _pass_c(h2d, st_b, gb, tm, fm)
    y2d = _pass_d(h2d, x16, w3, ws, st_a, st_b, st_c, gb, x.dtype, tm, fm)
    return y2d.reshape(n, h, w, ce)


# speedup vs baseline: 1.0540x; 1.0540x over previous
import functools

import jax
import jax.numpy as jnp
from jax import lax
from jax.experimental import pallas as pl
from jax.experimental.pallas import tpu as pltpu

_EPS = 1e-5
_VMEM_LIMIT = 64 * 1024 * 1024
_GRAM_DN = (((0,), (0,)), ((), ()))


def _cparams():
    return pltpu.CompilerParams(
        dimension_semantics=("parallel",),
        vmem_limit_bytes=_VMEM_LIMIT)


def _rup8(v):
    return ((v + 7) // 8) * 8


def _ik_gram_affine(gsum, usum, wmat, gamma, beta, count):
    mean = jnp.dot(usum, wmat, preferred_element_type=jnp.float32) / count
    gw = jnp.dot(gsum, wmat, preferred_element_type=jnp.float32)
    q = jnp.sum(wmat * gw, axis=0, keepdims=True) / count
    var = jnp.maximum(q - mean * mean, 0.0)
    scale = gamma * lax.rsqrt(var + _EPS)
    shift = beta - mean * scale
    return scale, shift


def _ik_direct_affine(ssum, qsum, gamma, beta, count):
    mean = ssum / count
    var = jnp.maximum(qsum / count - mean * mean, 0.0)
    scale = gamma * lax.rsqrt(var + _EPS)
    shift = beta - mean * scale
    return scale, shift


def _pass_a_kernel(x_ref, w1_ref, h_ref, sa_ref):
    xb = x_ref[...]
    x16 = xb.astype(jnp.bfloat16)
    w116 = w1_ref[...].astype(jnp.bfloat16)
    h = jnp.dot(x16, w116, preferred_element_type=jnp.float32)
    h_ref[...] = h.astype(jnp.bfloat16)
    g = lax.dot_general(x16, x16, _GRAM_DN,
                        preferred_element_type=jnp.float32)
    u = jnp.sum(xb, axis=0, keepdims=True)
    sa_ref[...] = jnp.concatenate([g, u], axis=0)[None]


def _pass_a(x2d, w1, tm):
    m, c0 = x2d.shape
    cm = w1.shape[1]
    nt = m // tm
    return pl.pallas_call(
        _pass_a_kernel,
        grid=(nt,),
        in_specs=[pl.BlockSpec((tm, c0), lambda i: (i, 0)),
                  pl.BlockSpec((c0, cm), lambda i: (0, 0))],
        out_specs=(pl.BlockSpec((tm, cm), lambda i: (i, 0)),
                   pl.BlockSpec((1, c0 + 1, c0), lambda i: (i, 0, 0))),
        out_shape=(jax.ShapeDtypeStruct((m, cm), jnp.bfloat16),
                   jax.ShapeDtypeStruct((nt, c0 + 1, c0), jnp.float32)),
        compiler_params=_cparams(),
        cost_estimate=pl.CostEstimate(
            flops=2 * m * c0 * (cm + c0), transcendentals=0,
            bytes_accessed=4 * m * c0 + 2 * m * cm),
    )(x2d, w1)


def _pass_b_kernel(x_ref, w_ref, sa_ref, w1_ref, gb_ref,
                   y_ref, sb_ref, p_ref, q1_ref, q7_ref, col_ref, *, count):
    nb, h, w, c = x_ref.shape
    c0 = w1_ref.shape[0]
    wp = _rup8(w + 2)
    base = wp + 8
    hw = h * wp
    psz = p_ref.shape[0] // nb
    qsz = q1_ref.shape[0] // nb
    cout = w_ref.shape[1]

    ta = jnp.sum(sa_ref[...], axis=0)
    sc1, sh1 = _ik_gram_affine(ta[:c0], ta[c0:c0 + 1], w1_ref[...],
                               gb_ref[0:1, :c], gb_ref[1:2, :c], count)

    for j in range(nb):
        a = jnp.maximum(x_ref[j].astype(jnp.float32) * sc1[0] + sh1[0], 0.0)
        pb = j * psz
        p_ref[pb:pb + base, :] = jnp.zeros((base, c), jnp.float32)
        tail = base + (h - 1) * wp + w
        p_ref[pb + tail:pb + psz, :] = jnp.zeros((psz - tail, c),
                                                 jnp.float32)
        zgap = jnp.zeros((wp - w, c), jnp.float32)
        for hh in range(h - 1):
            p_ref[pb + base + hh * wp + w:pb + base + (hh + 1) * wp, :] = \
                zgap
        for hh in range(h):
            p_ref[pb + base + hh * wp:pb + base + hh * wp + w, :] = a[hh]

        q1_ref[j * qsz:(j + 1) * qsz, :] = p_ref[pb + 1:pb + qsz + 1, :]
        q7_ref[j * qsz:(j + 1) * qsz, :] = p_ref[pb + 7:pb + qsz + 7, :]

        for kw in range(3):
            src = (q7_ref, p_ref, q1_ref)[kw]
            sb = (j * qsz, pb, j * qsz)[kw]
            start = sb + base - wp + (kw - 1) - (7, 0, 1)[kw]
            v = src[start:start + hw + 2 * wp, :]
            for kh in range(3):
                t = kh * 3 + kw
                col_ref[j * hw:(j + 1) * hw, t * c:(t + 1) * c] = \
                    v[kh * wp:kh * wp + hw].astype(jnp.bfloat16)

    y = jnp.dot(col_ref[...], w_ref[...].astype(jnp.bfloat16),
                preferred_element_type=jnp.float32)
    y4 = y.reshape(nb, h, wp, cout)[:, :, :w, :]
    y_ref[...] = y4.astype(jnp.bfloat16)
    sq = []
    for j in range(nb):
        yf = y4[j].reshape(h * w, cout)
        s = jnp.sum(yf, axis=0, keepdims=True)
        q = jnp.sum(yf * yf, axis=0, keepdims=True)
        sq.append(jnp.concatenate([s, q], axis=0))
    sb_ref[...] = jnp.stack(sq, axis=0)


def _pass_b(x4d, w2f, st_a, w1, gb, count):
    n, h, w, c = x4d.shape
    cout = w2f.shape[1]
    nta, c0p1, c0 = st_a.shape
    nb = 2 if n % 2 == 0 else 1
    wp = _rup8(w + 2)
    base = wp + 8
    hw = h * wp
    qsz = base + wp + hw
    psz = qsz + 8
    return pl.pallas_call(
        functools.partial(_pass_b_kernel, count=count),
        grid=(n // nb,),
        in_specs=[pl.BlockSpec((nb, h, w, c), lambda i: (i, 0, 0, 0)),
                  pl.BlockSpec((9 * c, cout), lambda i: (0, 0)),
                  pl.BlockSpec((nta, c0p1, c0), lambda i: (0, 0, 0)),
                  pl.BlockSpec((c0, c), lambda i: (0, 0)),
                  pl.BlockSpec(gb.shape, lambda i: (0, 0))],
        out_specs=(pl.BlockSpec((nb, h, w, cout), lambda i: (i, 0, 0, 0)),
                   pl.BlockSpec((nb, 2, cout), lambda i: (i, 0, 0))),
        out_shape=(jax.ShapeDtypeStruct((n, h, w, cout), jnp.bfloat16),
                   jax.ShapeDtypeStruct((n, 2, cout), jnp.float32)),
        scratch_shapes=[pltpu.VMEM((nb * psz, c), jnp.float32),
                        pltpu.VMEM((nb * qsz, c), jnp.float32),
                        pltpu.VMEM((nb * qsz, c), jnp.float32),
                        pltpu.VMEM((nb * hw, 9 * c), jnp.bfloat16)],
        compiler_params=_cparams(),
        cost_estimate=pl.CostEstimate(
            flops=2 * n * hw * 9 * c * cout, transcendentals=0,
            bytes_accessed=2 * (n * h * w * c + n * h * w * cout)
                           + 4 * 9 * c * cout),
    )(x4d, w2f, st_a, w1, gb)


def _pass_c_kernel(h2_ref, sb_ref, gb_ref, st_ref, *, count):
    c = h2_ref.shape[1]
    tb = jnp.sum(sb_ref[...], axis=0)
    sc2, sh2 = _ik_direct_affine(tb[0:1], tb[1:2],
                                 gb_ref[2:3, :c], gb_ref[3:4, :c], count)
    t = jnp.maximum(h2_ref[...].astype(jnp.float32) * sc2 + sh2, 0.0)
    t16 = t.astype(jnp.bfloat16)
    g = lax.dot_general(t16, t16, _GRAM_DN,
                        preferred_element_type=jnp.float32)
    u = jnp.sum(t, axis=0, keepdims=True)
    st_ref[...] = jnp.concatenate([g, u], axis=0)[None]


def _pass_c(h2d, st_b, gb, tm, count):
    m, cm = h2d.shape
    nt = m // tm
    n2, two, cmb = st_b.shape
    return pl.pallas_call(
        functools.partial(_pass_c_kernel, count=count),
        grid=(nt,),
        in_specs=[pl.BlockSpec((tm, cm), lambda i: (i, 0)),
                  pl.BlockSpec((n2, two, cmb), lambda i: (0, 0, 0)),
                  pl.BlockSpec(gb.shape, lambda i: (0, 0))],
        out_specs=pl.BlockSpec((1, cm + 1, cm), lambda i: (i, 0, 0)),
        out_shape=jax.ShapeDtypeStruct((nt, cm + 1, cm), jnp.float32),
        compiler_params=_cparams(),
        cost_estimate=pl.CostEstimate(
            flops=2 * m * cm * cm, transcendentals=0,
            bytes_accessed=2 * m * cm),
    )(h2d, st_b, gb)


def _pass_d_kernel(h2_ref, x_ref, w3_ref, ws_ref, sa_ref, sb_ref, st_ref,
                   gb_ref, o_ref, *, count):
    cm = w3_ref.shape[0]
    c0 = ws_ref.shape[0]
    tb = jnp.sum(sb_ref[...], axis=0)
    sc2, sh2 = _ik_direct_affine(tb[0:1], tb[1:2],
                                 gb_ref[2:3, :cm], gb_ref[3:4, :cm], count)
    w3f = w3_ref[...]
    wsf = ws_ref[...]
    tc = jnp.sum(st_ref[...], axis=0)
    sc3, sh3 = _ik_gram_affine(tc[:cm], tc[cm:cm + 1], w3f,
                               gb_ref[4:5, :], gb_ref[5:6, :], count)
    ta = jnp.sum(sa_ref[...], axis=0)
    scs, shs = _ik_gram_affine(ta[:c0], ta[c0:c0 + 1], wsf,
                               gb_ref[6:7, :], gb_ref[7:8, :], count)

    t = jnp.maximum(h2_ref[...].astype(jnp.float32) * sc2 + sh2, 0.0)
    z = jnp.dot(t.astype(jnp.bfloat16), w3f.astype(jnp.bfloat16),
                preferred_element_type=jnp.float32)
    r = jnp.dot(x_ref[...].astype(jnp.bfloat16), wsf.astype(jnp.bfloat16),
                preferred_element_type=jnp.float32)
    o = (z * sc3 + sh3) + (r * scs + shs)
    o_ref[...] = jnp.maximum(o, 0.0).astype(o_ref.dtype)


def _pass_d(h2d, x2d, w3, ws, st_a, st_b, st_c, gb, out_dtype, tm, count):
    m, cm = h2d.shape
    c0 = x2d.shape[1]
    ce = w3.shape[1]
    nt = m // tm
    nta, ap, ac = st_a.shape
    nb, two, cmb = st_b.shape
    ntc, cp, cc = st_c.shape
    return pl.pallas_call(
        functools.partial(_pass_d_kernel, count=count),
        grid=(nt,),
        in_specs=[pl.BlockSpec((tm, cm), lambda i: (i, 0)),
                  pl.BlockSpec((tm, c0), lambda i: (i, 0)),
                  pl.BlockSpec((cm, ce), lambda i: (0, 0)),
                  pl.BlockSpec((c0, ce), lambda i: (0, 0)),
                  pl.BlockSpec((nta, ap, ac), lambda i: (0, 0, 0)),
                  pl.BlockSpec((nb, two, cmb), lambda i: (0, 0, 0)),
                  pl.BlockSpec((ntc, cp, cc), lambda i: (0, 0, 0)),
                  pl.BlockSpec(gb.shape, lambda i: (0, 0))],
        out_specs=pl.BlockSpec((tm, ce), lambda i: (i, 0)),
        out_shape=jax.ShapeDtypeStruct((m, ce), out_dtype),
        compiler_params=_cparams(),
        cost_estimate=pl.CostEstimate(
            flops=2 * m * (cm + c0) * ce, transcendentals=0,
            bytes_accessed=2 * m * cm + 4 * m * c0 + 4 * m * ce),
    )(h2d, x2d, w3, ws, st_a, st_b, st_c, gb)


def kernel(x, w1, g1, b1, w2, g2, b2, w3, g3, b3, ws, gs, bs):
    n, h, w, c0 = x.shape
    cm = w1.shape[1]
    ce = w3.shape[1]
    m = n * h * w
    fm = float(m)
    tm = 16384 if m % 16384 == 0 else m

    x2d = x.reshape(m, c0)
    w2f = w2.reshape(9 * cm, cm)

    pad = lambda v: jnp.pad(v.reshape(1, -1),
                            ((0, 0), (0, ce - v.shape[-1])))
    gb = jnp.concatenate([pad(g1), pad(b1), pad(g2), pad(b2),
                          g3.reshape(1, -1), b3.reshape(1, -1),
                          gs.reshape(1, -1), bs.reshape(1, -1)], axis=0)

    h1, st_a = _pass_a(x2d, w1, tm)
    h2, st_b = _pass_b(h1.reshape(n, h, w, cm), w2f, st_a, w1, gb, fm)
    h2d = h2.reshape(m, cm)
    st_c = _pass_c(h2d, st_b, gb, tm, fm)
    y2d = _pass_d(h2d, x2d, w3, ws, st_a, st_b, st_c, gb, x.dtype, tm, fm)
    return y2d.reshape(n, h, w, ce)


# speedup vs baseline: 1.0973x; 1.0411x over previous
import functools

import jax
import jax.numpy as jnp
from jax import lax
from jax.experimental import pallas as pl
from jax.experimental.pallas import tpu as pltpu

_EPS = 1e-5
_VMEM_LIMIT = 64 * 1024 * 1024
_GRAM_DN = (((0,), (0,)), ((), ()))


def _cparams():
    return pltpu.CompilerParams(
        dimension_semantics=("parallel",),
        vmem_limit_bytes=_VMEM_LIMIT)


def _rup8(v):
    return ((v + 7) // 8) * 8


def _ik_gram_affine(gsum, usum, wmat, gamma, beta, count):
    mean = jnp.dot(usum, wmat, preferred_element_type=jnp.float32) / count
    gw = jnp.dot(gsum, wmat, preferred_element_type=jnp.float32)
    q = jnp.sum(wmat * gw, axis=0, keepdims=True) / count
    var = jnp.maximum(q - mean * mean, 0.0)
    scale = gamma * lax.rsqrt(var + _EPS)
    shift = beta - mean * scale
    return scale, shift


def _ik_direct_affine(ssum, qsum, gamma, beta, count):
    mean = ssum / count
    var = jnp.maximum(qsum / count - mean * mean, 0.0)
    scale = gamma * lax.rsqrt(var + _EPS)
    shift = beta - mean * scale
    return scale, shift


def _pass_a_kernel(x_ref, w1_ref, h_ref, sa_ref):
    xb = x_ref[...]
    x16 = xb.astype(jnp.bfloat16)
    w116 = w1_ref[...].astype(jnp.bfloat16)
    h = jnp.dot(x16, w116, preferred_element_type=jnp.float32)
    h_ref[...] = h.astype(jnp.bfloat16)
    g = lax.dot_general(x16, x16, _GRAM_DN,
                        preferred_element_type=jnp.float32)
    u = jnp.sum(xb, axis=0, keepdims=True)
    sa_ref[...] = jnp.concatenate([g, u], axis=0)[None]


def _pass_a(x2d, w1, tm):
    m, c0 = x2d.shape
    cm = w1.shape[1]
    nt = m // tm
    return pl.pallas_call(
        _pass_a_kernel,
        grid=(nt,),
        in_specs=[pl.BlockSpec((tm, c0), lambda i: (i, 0)),
                  pl.BlockSpec((c0, cm), lambda i: (0, 0))],
        out_specs=(pl.BlockSpec((tm, cm), lambda i: (i, 0)),
                   pl.BlockSpec((1, c0 + 1, c0), lambda i: (i, 0, 0))),
        out_shape=(jax.ShapeDtypeStruct((m, cm), jnp.bfloat16),
                   jax.ShapeDtypeStruct((nt, c0 + 1, c0), jnp.float32)),
        compiler_params=_cparams(),
        cost_estimate=pl.CostEstimate(
            flops=2 * m * c0 * (cm + c0), transcendentals=0,
            bytes_accessed=4 * m * c0 + 2 * m * cm),
    )(x2d, w1)


def _pass_b_kernel(x_ref, w_ref, sa_ref, w1_ref, gb_ref,
                   y_ref, sb_ref, p_ref, q1_ref, q7_ref, col_ref, *, count):
    nb, h, w, c = x_ref.shape
    c0 = w1_ref.shape[0]
    wp = _rup8(w + 2)
    base = wp + 8
    hw = h * wp
    psz = p_ref.shape[0] // nb
    qsz = q1_ref.shape[0] // nb
    cout = w_ref.shape[1]

    ta = jnp.sum(sa_ref[...], axis=0)
    sc1, sh1 = _ik_gram_affine(ta[:c0], ta[c0:c0 + 1], w1_ref[...],
                               gb_ref[0:1, :c], gb_ref[1:2, :c], count)

    for j in range(nb):
        a = jnp.maximum(x_ref[j].astype(jnp.float32) * sc1[0] + sh1[0], 0.0)
        pb = j * psz
        p_ref[pb:pb + base, :] = jnp.zeros((base, c), jnp.float32)
        tail = base + (h - 1) * wp + w
        p_ref[pb + tail:pb + psz, :] = jnp.zeros((psz - tail, c),
                                                 jnp.float32)
        zgap = jnp.zeros((wp - w, c), jnp.float32)
        for hh in range(h - 1):
            p_ref[pb + base + hh * wp + w:pb + base + (hh + 1) * wp, :] = \
                zgap
        for hh in range(h):
            p_ref[pb + base + hh * wp:pb + base + hh * wp + w, :] = a[hh]

        q1_ref[j * qsz:(j + 1) * qsz, :] = p_ref[pb + 1:pb + qsz + 1, :]
        q7_ref[j * qsz:(j + 1) * qsz, :] = p_ref[pb + 7:pb + qsz + 7, :]

        for kh in range(3):
            for kw in range(3):
                t = kh * 3 + kw
                src = (q7_ref, p_ref, q1_ref)[kw]
                sb = (j * qsz, pb, j * qsz)[kw]
                off = sb + base + (kh - 1) * wp + (kw - 1) - (7, 0, 1)[kw]
                col_ref[j * hw:(j + 1) * hw, t * c:(t + 1) * c] = (
                    src[off:off + hw, :].astype(jnp.bfloat16))

    y = jnp.dot(col_ref[...], w_ref[...].astype(jnp.bfloat16),
                preferred_element_type=jnp.float32)
    y4 = y.reshape(nb, h, wp, cout)[:, :, :w, :]
    y_ref[...] = y4.astype(jnp.bfloat16)
    sq = []
    for j in range(nb):
        yf = y4[j].reshape(h * w, cout)
        s = jnp.sum(yf, axis=0, keepdims=True)
        q = jnp.sum(yf * yf, axis=0, keepdims=True)
        sq.append(jnp.concatenate([s, q], axis=0))
    sb_ref[...] = jnp.stack(sq, axis=0)


def _pass_b(x4d, w2f, st_a, w1, gb, count):
    n, h, w, c = x4d.shape
    cout = w2f.shape[1]
    nta, c0p1, c0 = st_a.shape
    nb = 2 if n % 2 == 0 else 1
    wp = _rup8(w + 2)
    base = wp + 8
    hw = h * wp
    qsz = base + wp + hw
    psz = qsz + 8
    return pl.pallas_call(
        functools.partial(_pass_b_kernel, count=count),
        grid=(n // nb,),
        in_specs=[pl.BlockSpec((nb, h, w, c), lambda i: (i, 0, 0, 0)),
                  pl.BlockSpec((9 * c, cout), lambda i: (0, 0)),
                  pl.BlockSpec((nta, c0p1, c0), lambda i: (0, 0, 0)),
                  pl.BlockSpec((c0, c), lambda i: (0, 0)),
                  pl.BlockSpec(gb.shape, lambda i: (0, 0))],
        out_specs=(pl.BlockSpec((nb, h, w, cout), lambda i: (i, 0, 0, 0)),
                   pl.BlockSpec((nb, 2, cout), lambda i: (i, 0, 0))),
        out_shape=(jax.ShapeDtypeStruct((n, h, w, cout), jnp.bfloat16),
                   jax.ShapeDtypeStruct((n, 2, cout), jnp.float32)),
        scratch_shapes=[pltpu.VMEM((nb * psz, c), jnp.float32),
                        pltpu.VMEM((nb * qsz, c), jnp.float32),
                        pltpu.VMEM((nb * qsz, c), jnp.float32),
                        pltpu.VMEM((nb * hw, 9 * c), jnp.bfloat16)],
        compiler_params=_cparams(),
        cost_estimate=pl.CostEstimate(
            flops=2 * n * hw * 9 * c * cout, transcendentals=0,
            bytes_accessed=2 * (n * h * w * c + n * h * w * cout)
                           + 4 * 9 * c * cout),
    )(x4d, w2f, st_a, w1, gb)


def _pass_c_kernel(h2_ref, sb_ref, gb_ref, st_ref, *, count):
    c = h2_ref.shape[1]
    tb = jnp.sum(sb_ref[...], axis=0)
    sc2, sh2 = _ik_direct_affine(tb[0:1], tb[1:2],
                                 gb_ref[2:3, :c], gb_ref[3:4, :c], count)
    t = jnp.maximum(h2_ref[...].astype(jnp.float32) * sc2 + sh2, 0.0)
    t16 = t.astype(jnp.bfloat16)
    g = lax.dot_general(t16, t16, _GRAM_DN,
                        preferred_element_type=jnp.float32)
    u = jnp.sum(t, axis=0, keepdims=True)
    st_ref[...] = jnp.concatenate([g, u], axis=0)[None]


def _pass_c(h2d, st_b, gb, tm, count):
    m, cm = h2d.shape
    nt = m // tm
    n2, two, cmb = st_b.shape
    return pl.pallas_call(
        functools.partial(_pass_c_kernel, count=count),
        grid=(nt,),
        in_specs=[pl.BlockSpec((tm, cm), lambda i: (i, 0)),
                  pl.BlockSpec((n2, two, cmb), lambda i: (0, 0, 0)),
                  pl.BlockSpec(gb.shape, lambda i: (0, 0))],
        out_specs=pl.BlockSpec((1, cm + 1, cm), lambda i: (i, 0, 0)),
        out_shape=jax.ShapeDtypeStruct((nt, cm + 1, cm), jnp.float32),
        compiler_params=_cparams(),
        cost_estimate=pl.CostEstimate(
            flops=2 * m * cm * cm, transcendentals=0,
            bytes_accessed=2 * m * cm),
    )(h2d, st_b, gb)


def _pass_d_kernel(h2_ref, x_ref, w3_ref, ws_ref, sa_ref, sb_ref, st_ref,
                   gb_ref, o_ref, *, count):
    cm = w3_ref.shape[0]
    c0 = ws_ref.shape[0]
    tb = jnp.sum(sb_ref[...], axis=0)
    sc2, sh2 = _ik_direct_affine(tb[0:1], tb[1:2],
                                 gb_ref[2:3, :cm], gb_ref[3:4, :cm], count)
    w3f = w3_ref[...]
    wsf = ws_ref[...]
    tc = jnp.sum(st_ref[...], axis=0)
    sc3, sh3 = _ik_gram_affine(tc[:cm], tc[cm:cm + 1], w3f,
                               gb_ref[4:5, :], gb_ref[5:6, :], count)
    ta = jnp.sum(sa_ref[...], axis=0)
    scs, shs = _ik_gram_affine(ta[:c0], ta[c0:c0 + 1], wsf,
                               gb_ref[6:7, :], gb_ref[7:8, :], count)

    t = jnp.maximum(h2_ref[...].astype(jnp.float32) * sc2 + sh2, 0.0)
    z = jnp.dot(t.astype(jnp.bfloat16), w3f.astype(jnp.bfloat16),
                preferred_element_type=jnp.float32)
    r = jnp.dot(x_ref[...].astype(jnp.bfloat16), wsf.astype(jnp.bfloat16),
                preferred_element_type=jnp.float32)
    o = (z * sc3 + sh3) + (r * scs + shs)
    o_ref[...] = jnp.maximum(o, 0.0).astype(o_ref.dtype)


def _pass_d(h2d, x2d, w3, ws, st_a, st_b, st_c, gb, out_dtype, tm, count):
    m, cm = h2d.shape
    c0 = x2d.shape[1]
    ce = w3.shape[1]
    nt = m // tm
    nta, ap, ac = st_a.shape
    nb, two, cmb = st_b.shape
    ntc, cp, cc = st_c.shape
    return pl.pallas_call(
        functools.partial(_pass_d_kernel, count=count),
        grid=(nt,),
        in_specs=[pl.BlockSpec((tm, cm), lambda i: (i, 0)),
                  pl.BlockSpec((tm, c0), lambda i: (i, 0)),
                  pl.BlockSpec((cm, ce), lambda i: (0, 0)),
                  pl.BlockSpec((c0, ce), lambda i: (0, 0)),
                  pl.BlockSpec((nta, ap, ac), lambda i: (0, 0, 0)),
                  pl.BlockSpec((nb, two, cmb), lambda i: (0, 0, 0)),
                  pl.BlockSpec((ntc, cp, cc), lambda i: (0, 0, 0)),
                  pl.BlockSpec(gb.shape, lambda i: (0, 0))],
        out_specs=pl.BlockSpec((tm, ce), lambda i: (i, 0)),
        out_shape=jax.ShapeDtypeStruct((m, ce), out_dtype),
        compiler_params=_cparams(),
        cost_estimate=pl.CostEstimate(
            flops=2 * m * (cm + c0) * ce, transcendentals=0,
            bytes_accessed=2 * m * cm + 4 * m * c0 + 4 * m * ce),
    )(h2d, x2d, w3, ws, st_a, st_b, st_c, gb)


def kernel(x, w1, g1, b1, w2, g2, b2, w3, g3, b3, ws, gs, bs):
    n, h, w, c0 = x.shape
    cm = w1.shape[1]
    ce = w3.shape[1]
    m = n * h * w
    fm = float(m)
    tm = 16384 if m % 16384 == 0 else m

    x2d = x.reshape(m, c0)
    w2f = w2.reshape(9 * cm, cm)

    pad = lambda v: jnp.pad(v.reshape(1, -1),
                            ((0, 0), (0, ce - v.shape[-1])))
    gb = jnp.concatenate([pad(g1), pad(b1), pad(g2), pad(b2),
                          g3.reshape(1, -1), b3.reshape(1, -1),
                          gs.reshape(1, -1), bs.reshape(1, -1)], axis=0)

    h1, st_a = _pass_a(x2d, w1, tm)
    h2, st_b = _pass_b(h1.reshape(n, h, w, cm), w2f, st_a, w1, gb, fm)
    h2d = h2.reshape(m, cm)
    st_c = _pass_c(h2d, st_b, gb, tm, fm)
    y2d = _pass_d(h2d, x2d, w3, ws, st_a, st_b, st_c, gb, x.dtype, tm, fm)
    return y2d.reshape(n, h, w, ce)


# speedup vs baseline: 1.1084x; 1.0101x over previous
import functools

import jax
import jax.numpy as jnp
from jax import lax
from jax.experimental import pallas as pl
from jax.experimental.pallas import tpu as pltpu

_EPS = 1e-5
_VMEM_LIMIT = 64 * 1024 * 1024
_GRAM_DN = (((0,), (0,)), ((), ()))


def _cparams():
    return pltpu.CompilerParams(
        dimension_semantics=("parallel",),
        vmem_limit_bytes=_VMEM_LIMIT)


def _rup8(v):
    return ((v + 7) // 8) * 8


def _ik_gram_affine(gsum, usum, wmat, gamma, beta, count):
    mean = jnp.dot(usum, wmat, preferred_element_type=jnp.float32) / count
    gw = jnp.dot(gsum, wmat, preferred_element_type=jnp.float32)
    q = jnp.sum(wmat * gw, axis=0, keepdims=True) / count
    var = jnp.maximum(q - mean * mean, 0.0)
    scale = gamma * lax.rsqrt(var + _EPS)
    shift = beta - mean * scale
    return scale, shift


def _ik_direct_affine(ssum, qsum, gamma, beta, count):
    mean = ssum / count
    var = jnp.maximum(qsum / count - mean * mean, 0.0)
    scale = gamma * lax.rsqrt(var + _EPS)
    shift = beta - mean * scale
    return scale, shift


def _pass_a_kernel(x_ref, w1_ref, h_ref, x16_ref, sa_ref):
    xb = x_ref[...]
    x16 = xb.astype(jnp.bfloat16)
    x16_ref[...] = x16
    w116 = w1_ref[...].astype(jnp.bfloat16)
    h = jnp.dot(x16, w116, preferred_element_type=jnp.float32)
    h_ref[...] = h.astype(jnp.bfloat16)
    g = lax.dot_general(x16, x16, _GRAM_DN,
                        preferred_element_type=jnp.float32)
    u = jnp.sum(xb, axis=0, keepdims=True)
    sa_ref[...] = jnp.concatenate([g, u], axis=0)[None]


def _pass_a(x2d, w1, tm):
    m, c0 = x2d.shape
    cm = w1.shape[1]
    nt = m // tm
    return pl.pallas_call(
        _pass_a_kernel,
        grid=(nt,),
        in_specs=[pl.BlockSpec((tm, c0), lambda i: (i, 0)),
                  pl.BlockSpec((c0, cm), lambda i: (0, 0))],
        out_specs=(pl.BlockSpec((tm, cm), lambda i: (i, 0)),
                   pl.BlockSpec((tm, c0), lambda i: (i, 0)),
                   pl.BlockSpec((1, c0 + 1, c0), lambda i: (i, 0, 0))),
        out_shape=(jax.ShapeDtypeStruct((m, cm), jnp.bfloat16),
                   jax.ShapeDtypeStruct((m, c0), jnp.bfloat16),
                   jax.ShapeDtypeStruct((nt, c0 + 1, c0), jnp.float32)),
        compiler_params=_cparams(),
        cost_estimate=pl.CostEstimate(
            flops=2 * m * c0 * (cm + c0), transcendentals=0,
            bytes_accessed=4 * m * c0 + 2 * m * cm + 2 * m * c0),
    )(x2d, w1)


def _pass_b_kernel(x_ref, w_ref, sa_ref, w1_ref, gb_ref,
                   y_ref, sb_ref, p_ref, q1_ref, q7_ref, col_ref, *, count):
    nb, h, w, c = x_ref.shape
    c0 = w1_ref.shape[0]
    wp = _rup8(w + 2)
    base = wp + 8
    hw = h * wp
    psz = p_ref.shape[0] // nb
    qsz = q1_ref.shape[0] // nb
    cout = w_ref.shape[1]

    ta = jnp.sum(sa_ref[...], axis=0)
    sc1, sh1 = _ik_gram_affine(ta[:c0], ta[c0:c0 + 1], w1_ref[...],
                               gb_ref[0:1, :c], gb_ref[1:2, :c], count)

    for j in range(nb):
        a = jnp.maximum(x_ref[j].astype(jnp.float32) * sc1[0] + sh1[0], 0.0)
        pb = j * psz
        p_ref[pb:pb + base, :] = jnp.zeros((base, c), jnp.float32)
        tail = base + (h - 1) * wp + w
        p_ref[pb + tail:pb + psz, :] = jnp.zeros((psz - tail, c),
                                                 jnp.float32)
        zgap = jnp.zeros((wp - w, c), jnp.float32)
        for hh in range(h - 1):
            p_ref[pb + base + hh * wp + w:pb + base + (hh + 1) * wp, :] = \
                zgap
        for hh in range(h):
            p_ref[pb + base + hh * wp:pb + base + hh * wp + w, :] = a[hh]

        q1_ref[j * qsz:(j + 1) * qsz, :] = p_ref[pb + 1:pb + qsz + 1, :]
        q7_ref[j * qsz:(j + 1) * qsz, :] = p_ref[pb + 7:pb + qsz + 7, :]

        for kh in range(3):
            for kw in range(3):
                t = kh * 3 + kw
                src = (q7_ref, p_ref, q1_ref)[kw]
                sb = (j * qsz, pb, j * qsz)[kw]
                off = sb + base + (kh - 1) * wp + (kw - 1) - (7, 0, 1)[kw]
                col_ref[j * hw:(j + 1) * hw, t * c:(t + 1) * c] = (
                    src[off:off + hw, :].astype(jnp.bfloat16))

    y = jnp.dot(col_ref[...], w_ref[...].astype(jnp.bfloat16),
                preferred_element_type=jnp.float32)
    y4 = y.reshape(nb, h, wp, cout)[:, :, :w, :]
    y_ref[...] = y4.astype(jnp.bfloat16)
    sq = []
    for j in range(nb):
        yf = y4[j].reshape(h * w, cout)
        s = jnp.sum(yf, axis=0, keepdims=True)
        q = jnp.sum(yf * yf, axis=0, keepdims=True)
        sq.append(jnp.concatenate([s, q], axis=0))
    sb_ref[...] = jnp.stack(sq, axis=0)


def _pass_b(x4d, w2f, st_a, w1, gb, count):
    n, h, w, c = x4d.shape
    cout = w2f.shape[1]
    nta, c0p1, c0 = st_a.shape
    nb = 2 if n % 2 == 0 else 1
    wp = _rup8(w + 2)
    base = wp + 8
    hw = h * wp
    qsz = base + wp + hw
    psz = qsz + 8
    return pl.pallas_call(
        functools.partial(_pass_b_kernel, count=count),
        grid=(n // nb,),
        in_specs=[pl.BlockSpec((nb, h, w, c), lambda i: (i, 0, 0, 0)),
                  pl.BlockSpec((9 * c, cout), lambda i: (0, 0)),
                  pl.BlockSpec((nta, c0p1, c0), lambda i: (0, 0, 0)),
                  pl.BlockSpec((c0, c), lambda i: (0, 0)),
                  pl.BlockSpec(gb.shape, lambda i: (0, 0))],
        out_specs=(pl.BlockSpec((nb, h, w, cout), lambda i: (i, 0, 0, 0)),
                   pl.BlockSpec((nb, 2, cout), lambda i: (i, 0, 0))),
        out_shape=(jax.ShapeDtypeStruct((n, h, w, cout), jnp.bfloat16),
                   jax.ShapeDtypeStruct((n, 2, cout), jnp.float32)),
        scratch_shapes=[pltpu.VMEM((nb * psz, c), jnp.float32),
                        pltpu.VMEM((nb * qsz, c), jnp.float32),
                        pltpu.VMEM((nb * qsz, c), jnp.float32),
                        pltpu.VMEM((nb * hw, 9 * c), jnp.bfloat16)],
        compiler_params=_cparams(),
        cost_estimate=pl.CostEstimate(
            flops=2 * n * hw * 9 * c * cout, transcendentals=0,
            bytes_accessed=2 * (n * h * w * c + n * h * w * cout)
                           + 4 * 9 * c * cout),
    )(x4d, w2f, st_a, w1, gb)


def _pass_c_kernel(h2_ref, sb_ref, gb_ref, st_ref, *, count):
    c = h2_ref.shape[1]
    tb = jnp.sum(sb_ref[...], axis=0)
    sc2, sh2 = _ik_direct_affine(tb[0:1], tb[1:2],
                                 gb_ref[2:3, :c], gb_ref[3:4, :c], count)
    t = jnp.maximum(h2_ref[...].astype(jnp.float32) * sc2 + sh2, 0.0)
    t16 = t.astype(jnp.bfloat16)
    g = lax.dot_general(t16, t16, _GRAM_DN,
                        preferred_element_type=jnp.float32)
    u = jnp.sum(t, axis=0, keepdims=True)
    st_ref[...] = jnp.concatenate([g, u], axis=0)[None]


def _pass_c(h2d, st_b, gb, tm, count):
    m, cm = h2d.shape
    nt = m // tm
    n2, two, cmb = st_b.shape
    return pl.pallas_call(
        functools.partial(_pass_c_kernel, count=count),
        grid=(nt,),
        in_specs=[pl.BlockSpec((tm, cm), lambda i: (i, 0)),
                  pl.BlockSpec((n2, two, cmb), lambda i: (0, 0, 0)),
                  pl.BlockSpec(gb.shape, lambda i: (0, 0))],
        out_specs=pl.BlockSpec((1, cm + 1, cm), lambda i: (i, 0, 0)),
        out_shape=jax.ShapeDtypeStruct((nt, cm + 1, cm), jnp.float32),
        compiler_params=_cparams(),
        cost_estimate=pl.CostEstimate(
            flops=2 * m * cm * cm, transcendentals=0,
            bytes_accessed=2 * m * cm),
    )(h2d, st_b, gb)


def _pass_d_kernel(h2_ref, x_ref, w3_ref, ws_ref, sa_ref, sb_ref, st_ref,
                   gb_ref, o_ref, cat_ref, *, count):
    cm = w3_ref.shape[0]
    c0 = ws_ref.shape[0]
    tb = jnp.sum(sb_ref[...], axis=0)
    sc2, sh2 = _ik_direct_affine(tb[0:1], tb[1:2],
                                 gb_ref[2:3, :cm], gb_ref[3:4, :cm], count)
    w3f = w3_ref[...]
    wsf = ws_ref[...]
    tc = jnp.sum(st_ref[...], axis=0)
    sc3, sh3 = _ik_gram_affine(tc[:cm], tc[cm:cm + 1], w3f,
                               gb_ref[4:5, :], gb_ref[5:6, :], count)
    ta = jnp.sum(sa_ref[...], axis=0)
    scs, shs = _ik_gram_affine(ta[:c0], ta[c0:c0 + 1], wsf,
                               gb_ref[6:7, :], gb_ref[7:8, :], count)

    wcat = jnp.concatenate([w3f * sc3, wsf * scs], axis=0)
    t = jnp.maximum(h2_ref[...].astype(jnp.float32) * sc2 + sh2, 0.0)
    cat_ref[:, :cm] = t.astype(jnp.bfloat16)
    cat_ref[:, cm:] = x_ref[...]
    zr = jnp.dot(cat_ref[...], wcat.astype(jnp.bfloat16),
                 preferred_element_type=jnp.float32)
    o_ref[...] = jnp.maximum(zr + (sh3 + shs), 0.0).astype(o_ref.dtype)


def _pass_d(h2d, x2d, w3, ws, st_a, st_b, st_c, gb, out_dtype, tm, count):
    m, cm = h2d.shape
    c0 = x2d.shape[1]
    ce = w3.shape[1]
    nt = m // tm
    nta, ap, ac = st_a.shape
    nb, two, cmb = st_b.shape
    ntc, cp, cc = st_c.shape
    return pl.pallas_call(
        functools.partial(_pass_d_kernel, count=count),
        grid=(nt,),
        in_specs=[pl.BlockSpec((tm, cm), lambda i: (i, 0)),
                  pl.BlockSpec((tm, c0), lambda i: (i, 0)),
                  pl.BlockSpec((cm, ce), lambda i: (0, 0)),
                  pl.BlockSpec((c0, ce), lambda i: (0, 0)),
                  pl.BlockSpec((nta, ap, ac), lambda i: (0, 0, 0)),
                  pl.BlockSpec((nb, two, cmb), lambda i: (0, 0, 0)),
                  pl.BlockSpec((ntc, cp, cc), lambda i: (0, 0, 0)),
                  pl.BlockSpec(gb.shape, lambda i: (0, 0))],
        out_specs=pl.BlockSpec((tm, ce), lambda i: (i, 0)),
        out_shape=jax.ShapeDtypeStruct((m, ce), out_dtype),
        scratch_shapes=[pltpu.VMEM((tm, cm + c0), jnp.bfloat16)],
        compiler_params=_cparams(),
        cost_estimate=pl.CostEstimate(
            flops=2 * m * (cm + c0) * ce, transcendentals=0,
            bytes_accessed=2 * m * cm + 2 * m * c0 + 4 * m * ce),
    )(h2d, x2d, w3, ws, st_a, st_b, st_c, gb)


def kernel(x, w1, g1, b1, w2, g2, b2, w3, g3, b3, ws, gs, bs):
    n, h, w, c0 = x.shape
    cm = w1.shape[1]
    ce = w3.shape[1]
    m = n * h * w
    fm = float(m)
    tm = 16384 if m % 16384 == 0 else m

    x2d = x.reshape(m, c0)
    w2f = w2.reshape(9 * cm, cm)

    pad = lambda v: jnp.pad(v.reshape(1, -1),
                            ((0, 0), (0, ce - v.shape[-1])))
    gb = jnp.concatenate([pad(g1), pad(b1), pad(g2), pad(b2),
                          g3.reshape(1, -1), b3.reshape(1, -1),
                          gs.reshape(1, -1), bs.reshape(1, -1)], axis=0)

    h1, x16, st_a = _pass_a(x2d, w1, tm)
    h2, st_b = _pass_b(h1.reshape(n, h, w, cm), w2f, st_a, w1, gb, fm)
    h2d = h2.reshape(m, cm)
    st_c = _pass_c(h2d, st_b, gb, tm, fm)
    y2d = _pass_d(h2d, x16, w3, ws, st_a, st_b, st_c, gb, x.dtype, tm, fm)
    return y2d.reshape(n, h, w, ce)
